# baseline (speedup 1.0000x reference)
import functools

import jax
import jax.numpy as jnp
from jax import lax
from jax.experimental import pallas as pl
from jax.experimental.pallas import tpu as pltpu
from jax.experimental.pallas import tpu_sc as plsc

N_NODES = 10000
N_EDGES = 320000
D = 128

NC = 2
NS = 16
NW = NC * NS
E_PER_TILE = N_EDGES // NW
K = 100
CHUNKS = E_PER_TILE // K
HALVES = 2
HCHUNKS = CHUNKS // HALVES
KA = 48
KB = K - KA
ROWS_PER_TILE = 624
TAIL_ROW = NS * ROWS_PER_TILE
TAIL = N_NODES - TAIL_ROW


def _zero_acc(z128_hbm, acc_sh, s, row):
    pltpu.sync_copy(z128_hbm.at[pl.ds(row, ROWS_PER_TILE)],
                    acc_sh.at[pl.ds(row, ROWS_PER_TILE)])

    @pl.when(s == 0)
    def _():
        pltpu.sync_copy(z128_hbm.at[pl.ds(TAIL_ROW, TAIL)],
                        acc_sh.at[pl.ds(TAIL_ROW, TAIL)])


def _writeback(acc_sh, out_hbm, c, s, row):
    pltpu.sync_copy(acc_sh.at[pl.ds(row, ROWS_PER_TILE)],
                    out_hbm.at[c, pl.ds(row, ROWS_PER_TILE)])

    @pl.when(s == 0)
    def _():
        pltpu.sync_copy(acc_sh.at[pl.ds(TAIL_ROW, TAIL)],
                        out_hbm.at[c, pl.ds(TAIL_ROW, TAIL)])


def _sc_agg_body(h_hbm, src4_hbm, dst4_hbm, z128_hbm, ones_hbm,
                 sums_hbm, cnts_hbm,
                 acc_sh, src_all, dst_all, rows0, rows1,
                 sg0, sg0b, sg1, sg1b, ss0, ss1):
    c = lax.axis_index("c")
    s = lax.axis_index("s")
    wid = c * NS + s
    row = s * ROWS_PER_TILE

    _zero_acc(z128_hbm, acc_sh, s, row)
    plsc.subcore_barrier()

    for half in range(HALVES):
        pltpu.sync_copy(src4_hbm.at[wid, half], src_all)
        pltpu.sync_copy(dst4_hbm.at[wid, half], dst_all)

        def gather(j, rows, sa, sb):
            pltpu.async_copy(h_hbm.at[src_all.at[j, pl.ds(0, KA)]],
                             rows.at[pl.ds(0, KA)], sa)
            pltpu.async_copy(h_hbm.at[src_all.at[j, pl.ds(KA, KB)]],
                             rows.at[pl.ds(KA, KB)], sb)

        def gwait(j, rows, sa, sb):
            pltpu.make_async_copy(h_hbm.at[src_all.at[j, pl.ds(0, KA)]],
                                  rows.at[pl.ds(0, KA)], sa).wait()
            pltpu.make_async_copy(h_hbm.at[src_all.at[j, pl.ds(KA, KB)]],
                                  rows.at[pl.ds(KA, KB)], sb).wait()

        gather(0, rows0, sg0, sg0b)
        gather(1, rows1, sg1, sg1b)

        def body(g, carry):
            j = g * 2
            gwait(j, rows0, sg0, sg0b)
            pltpu.async_copy(rows0, acc_sh.at[dst_all.at[j]], ss0,
                             add=True)
            gwait(j + 1, rows1, sg1, sg1b)
            pltpu.async_copy(rows1, acc_sh.at[dst_all.at[j + 1]], ss1,
                             add=True)
            pltpu.make_async_copy(rows0, acc_sh.at[dst_all.at[j]],
                                  ss0).wait()
            gather(j + 2, rows0, sg0, sg0b)
            pltpu.make_async_copy(rows1, acc_sh.at[dst_all.at[j + 1]],
                                  ss1).wait()
            gather(j + 3, rows1, sg1, sg1b)
            return carry

        lax.fori_loop(0, HCHUNKS // 2 - 1, body, 0)

        j = HCHUNKS - 2
        gwait(j, rows0, sg0, sg0b)
        pltpu.async_copy(rows0, acc_sh.at[dst_all.at[j]], ss0, add=True)
        gwait(j + 1, rows1, sg1, sg1b)
        pltpu.async_copy(rows1, acc_sh.at[dst_all.at[j + 1]], ss1,
                         add=True)
        pltpu.make_async_copy(rows0, acc_sh.at[dst_all.at[j]],
                              ss0).wait()
        pltpu.make_async_copy(rows1, acc_sh.at[dst_all.at[j + 1]],
                              ss1).wait()
    plsc.subcore_barrier()

    _writeback(acc_sh, sums_hbm, c, s, row)
    plsc.subcore_barrier()

    _zero_acc(z128_hbm, acc_sh, s, row)
    pltpu.sync_copy(ones_hbm, rows0)
    plsc.subcore_barrier()

    for half in range(HALVES):
        pltpu.sync_copy(dst4_hbm.at[wid, half], dst_all)
        pltpu.async_copy(rows0, acc_sh.at[dst_all.at[0]], ss0, add=True)
        pltpu.async_copy(rows0, acc_sh.at[dst_all.at[1]], ss1, add=True)

        def cbody(g, carry):
            j = g * 2
            pltpu.make_async_copy(rows0, acc_sh.at[dst_all.at[j]],
                                  ss0).wait()
            pltpu.async_copy(rows0, acc_sh.at[dst_all.at[j + 2]], ss0,
                             add=True)
            pltpu.make_async_copy(rows0, acc_sh.at[dst_all.at[j + 1]],
                                  ss1).wait()
            pltpu.async_copy(rows0, acc_sh.at[dst_all.at[j + 3]], ss1,
                             add=True)
            return carry

        lax.fori_loop(0, HCHUNKS // 2 - 1, cbody, 0)
        pltpu.make_async_copy(rows0, acc_sh.at[dst_all.at[HCHUNKS - 2]],
                              ss0).wait()
        pltpu.make_async_copy(rows0, acc_sh.at[dst_all.at[HCHUNKS - 1]],
                              ss1).wait()
    plsc.subcore_barrier()

    _writeback(acc_sh, cnts_hbm, c, s, row)


@functools.cache
def _sc_kernel():
    mesh = plsc.VectorSubcoreMesh(core_axis_name="c", subcore_axis_name="s",
                                  num_cores=NC, num_subcores=NS)
    return pl.kernel(
        _sc_agg_body,
        out_type=(jax.ShapeDtypeStruct((NC, N_NODES, D), jnp.float32),
                  jax.ShapeDtypeStruct((NC, N_NODES, D), jnp.float32)),
        mesh=mesh,
        scratch_types=[
            pltpu.VMEM_SHARED((N_NODES, D), jnp.float32),
            pltpu.VMEM((HCHUNKS, K), jnp.int32),
            pltpu.VMEM((HCHUNKS, K), jnp.int32),
            pltpu.VMEM((K, D), jnp.float32),
            pltpu.VMEM((K, D), jnp.float32),
            pltpu.SemaphoreType.DMA,
            pltpu.SemaphoreType.DMA,
            pltpu.SemaphoreType.DMA,
            pltpu.SemaphoreType.DMA,
            pltpu.SemaphoreType.DMA,
            pltpu.SemaphoreType.DMA,
        ],
        name="sage_sc_aggregate",
    )


ROW_BLK = 1000
GRID = N_NODES // ROW_BLK


def _tc_dense_kernel(s_ref, c_ref, h_ref, wl_ref, b_ref, wr_ref, o_ref):
    summed = s_ref[0] + s_ref[1]
    cnt = c_ref[0, :, 0:1] + c_ref[1, :, 0:1]
    mean = summed / jnp.maximum(cnt, 1.0)
    h = h_ref[...]
    z = (jnp.dot(mean, wl_ref[...], preferred_element_type=jnp.float32)
         + jnp.dot(h, wr_ref[...], preferred_element_type=jnp.float32)
         + b_ref[...])
    o_ref[...] = jnp.where(z > 0, z, jnp.exp(jnp.minimum(z, 0.0)) - 1.0)


def _tc_dense(sums, cnts, h, W_l, b, W_r):
    return pl.pallas_call(
        _tc_dense_kernel,
        grid=(GRID,),
        in_specs=[
            pl.BlockSpec((NC, ROW_BLK, D), lambda i: (0, i, 0)),
            pl.BlockSpec((NC, ROW_BLK, D), lambda i: (0, i, 0)),
            pl.BlockSpec((ROW_BLK, D), lambda i: (i, 0)),
            pl.BlockSpec((D, D), lambda i: (0, 0)),
            pl.BlockSpec((1, D), lambda i: (0, 0)),
            pl.BlockSpec((D, D), lambda i: (0, 0)),
        ],
        out_specs=pl.BlockSpec((ROW_BLK, D), lambda i: (i, 0)),
        out_shape=jax.ShapeDtypeStruct((N_NODES, D), jnp.float32),
    )(sums, cnts, h, W_l, b, W_r)


def _tc_logsoftmax_kernel(h_ref, o_ref):
    z = h_ref[...]
    m = jnp.max(z, axis=1, keepdims=True)
    lse = m + jnp.log(jnp.sum(jnp.exp(z - m), axis=1, keepdims=True))
    o_ref[...] = z - lse


def _tc_logsoftmax(h):
    return pl.pallas_call(
        _tc_logsoftmax_kernel,
        grid=(GRID,),
        in_specs=[pl.BlockSpec((ROW_BLK, D), lambda i: (i, 0))],
        out_specs=pl.BlockSpec((ROW_BLK, D), lambda i: (i, 0)),
        out_shape=jax.ShapeDtypeStruct((N_NODES, D), jnp.float32),
    )(h)


def kernel(x, edge_index, W_l1, b1, W_r1, W_l2, b2, W_r2):
    src4 = edge_index[0].reshape(NW, HALVES, HCHUNKS, K)
    dst4 = edge_index[1].reshape(NW, HALVES, HCHUNKS, K)
    z128 = jnp.zeros((N_NODES, D), jnp.float32)
    ones128 = jnp.ones((K, D), jnp.float32)

    Wl_s = jnp.stack([W_l1, W_l2])
    b_s = jnp.stack([b1.reshape(1, D), b2.reshape(1, D)])
    Wr_s = jnp.stack([W_r1, W_r2])

    agg = _sc_kernel()

    def layer(h, w):
        Wl, b, Wr = w
        sums, cnts = agg(h, src4, dst4, z128, ones128)
        return _tc_dense(sums, cnts, h, Wl, b, Wr), None

    h2, _ = lax.scan(layer, x, (Wl_s, b_s, Wr_s))
    return _tc_logsoftmax(h2)


# speedup vs baseline: 1.1988x; 1.1988x over previous
import functools

import jax
import jax.numpy as jnp
from jax import lax
from jax.experimental import pallas as pl
from jax.experimental.pallas import tpu as pltpu
from jax.experimental.pallas import tpu_sc as plsc

N_NODES = 10000
N_EDGES = 320000
D = 128

NC = 2
NS = 16
NW = NC * NS
E_PER_TILE = N_EDGES // NW
K = 100
CHUNKS = E_PER_TILE // K
HALVES = 2
HCHUNKS = CHUNKS // HALVES
KA = 48
KB = K - KA
ROWS_PER_TILE = 624
TAIL_ROW = NS * ROWS_PER_TILE
TAIL = N_NODES - TAIL_ROW


def _zero_acc(z128_hbm, acc_sh, s, row):
    pltpu.sync_copy(z128_hbm.at[pl.ds(row, ROWS_PER_TILE)],
                    acc_sh.at[pl.ds(row, ROWS_PER_TILE)])

    @pl.when(s == 0)
    def _():
        pltpu.sync_copy(z128_hbm.at[pl.ds(TAIL_ROW, TAIL)],
                        acc_sh.at[pl.ds(TAIL_ROW, TAIL)])


def _writeback(acc_sh, out_hbm, c, s, row):
    pltpu.sync_copy(acc_sh.at[pl.ds(row, ROWS_PER_TILE)],
                    out_hbm.at[c, pl.ds(row, ROWS_PER_TILE)])

    @pl.when(s == 0)
    def _():
        pltpu.sync_copy(acc_sh.at[pl.ds(TAIL_ROW, TAIL)],
                        out_hbm.at[c, pl.ds(TAIL_ROW, TAIL)])


def _sc_agg_body(with_counts, h_hbm, src4_hbm, dst4_hbm, z128_hbm,
                 ones_hbm, sums_hbm, cnts_hbm,
                 acc_sh, src_all, dst_all, rows0, rows1,
                 sg0, sg0b, sg1, sg1b, ss0, ss1):
    c = lax.axis_index("c")
    s = lax.axis_index("s")
    wid = c * NS + s
    row = s * ROWS_PER_TILE

    _zero_acc(z128_hbm, acc_sh, s, row)
    plsc.subcore_barrier()

    for half in range(HALVES):
        pltpu.sync_copy(src4_hbm.at[wid, half], src_all)
        pltpu.sync_copy(dst4_hbm.at[wid, half], dst_all)

        def gather(j, rows, sa, sb):
            pltpu.async_copy(h_hbm.at[src_all.at[j, pl.ds(0, KA)]],
                             rows.at[pl.ds(0, KA)], sa)
            pltpu.async_copy(h_hbm.at[src_all.at[j, pl.ds(KA, KB)]],
                             rows.at[pl.ds(KA, KB)], sb)

        def gwait(j, rows, sa, sb):
            pltpu.make_async_copy(h_hbm.at[src_all.at[j, pl.ds(0, KA)]],
                                  rows.at[pl.ds(0, KA)], sa).wait()
            pltpu.make_async_copy(h_hbm.at[src_all.at[j, pl.ds(KA, KB)]],
                                  rows.at[pl.ds(KA, KB)], sb).wait()

        gather(0, rows0, sg0, sg0b)
        gather(1, rows1, sg1, sg1b)

        def body(g, carry):
            j = g * 2
            gwait(j, rows0, sg0, sg0b)
            pltpu.async_copy(rows0, acc_sh.at[dst_all.at[j]], ss0,
                             add=True)
            gwait(j + 1, rows1, sg1, sg1b)
            pltpu.async_copy(rows1, acc_sh.at[dst_all.at[j + 1]], ss1,
                             add=True)
            pltpu.make_async_copy(rows0, acc_sh.at[dst_all.at[j]],
                                  ss0).wait()
            gather(j + 2, rows0, sg0, sg0b)
            pltpu.make_async_copy(rows1, acc_sh.at[dst_all.at[j + 1]],
                                  ss1).wait()
            gather(j + 3, rows1, sg1, sg1b)
            return carry

        lax.fori_loop(0, HCHUNKS // 2 - 1, body, 0)

        j = HCHUNKS - 2
        gwait(j, rows0, sg0, sg0b)
        pltpu.async_copy(rows0, acc_sh.at[dst_all.at[j]], ss0, add=True)
        gwait(j + 1, rows1, sg1, sg1b)
        pltpu.async_copy(rows1, acc_sh.at[dst_all.at[j + 1]], ss1,
                         add=True)
        pltpu.make_async_copy(rows0, acc_sh.at[dst_all.at[j]],
                              ss0).wait()
        pltpu.make_async_copy(rows1, acc_sh.at[dst_all.at[j + 1]],
                              ss1).wait()
    plsc.subcore_barrier()

    _writeback(acc_sh, sums_hbm, c, s, row)
    if not with_counts:
        return
    plsc.subcore_barrier()

    _zero_acc(z128_hbm, acc_sh, s, row)
    pltpu.sync_copy(ones_hbm, rows0)
    plsc.subcore_barrier()

    for half in range(HALVES):
        pltpu.sync_copy(dst4_hbm.at[wid, half], dst_all)
        pltpu.async_copy(rows0, acc_sh.at[dst_all.at[0]], ss0, add=True)
        pltpu.async_copy(rows0, acc_sh.at[dst_all.at[1]], ss1, add=True)

        def cbody(g, carry):
            j = g * 2
            pltpu.make_async_copy(rows0, acc_sh.at[dst_all.at[j]],
                                  ss0).wait()
            pltpu.async_copy(rows0, acc_sh.at[dst_all.at[j + 2]], ss0,
                             add=True)
            pltpu.make_async_copy(rows0, acc_sh.at[dst_all.at[j + 1]],
                                  ss1).wait()
            pltpu.async_copy(rows0, acc_sh.at[dst_all.at[j + 3]], ss1,
                             add=True)
            return carry

        lax.fori_loop(0, HCHUNKS // 2 - 1, cbody, 0)
        pltpu.make_async_copy(rows0, acc_sh.at[dst_all.at[HCHUNKS - 2]],
                              ss0).wait()
        pltpu.make_async_copy(rows0, acc_sh.at[dst_all.at[HCHUNKS - 1]],
                              ss1).wait()
    plsc.subcore_barrier()

    _writeback(acc_sh, cnts_hbm, c, s, row)


@functools.cache
def _sc_kernel(with_counts):
    mesh = plsc.VectorSubcoreMesh(core_axis_name="c", subcore_axis_name="s",
                                  num_cores=NC, num_subcores=NS)
    if with_counts:
        body = functools.partial(_sc_agg_body, True)
    else:
        def body(h, s4, d4, z, o, sums, *scratch):
            _sc_agg_body(False, h, s4, d4, z, o, sums, None, *scratch)
    out_type = [jax.ShapeDtypeStruct((NC, N_NODES, D), jnp.float32),
                jax.ShapeDtypeStruct((NC, N_NODES, D), jnp.float32)]
    if not with_counts:
        out_type = out_type[:1]
    return pl.kernel(
        body,
        out_type=tuple(out_type),
        mesh=mesh,
        scratch_types=[
            pltpu.VMEM_SHARED((N_NODES, D), jnp.float32),
            pltpu.VMEM((HCHUNKS, K), jnp.int32),
            pltpu.VMEM((HCHUNKS, K), jnp.int32),
            pltpu.VMEM((K, D), jnp.float32),
            pltpu.VMEM((K, D), jnp.float32),
            pltpu.SemaphoreType.DMA,
            pltpu.SemaphoreType.DMA,
            pltpu.SemaphoreType.DMA,
            pltpu.SemaphoreType.DMA,
            pltpu.SemaphoreType.DMA,
            pltpu.SemaphoreType.DMA,
        ],
        name="sage_sc_aggregate_cnt" if with_counts else "sage_sc_aggregate",
    )


ROW_BLK = 1000
GRID = N_NODES // ROW_BLK


def _tc_dense_kernel(s_ref, c_ref, h_ref, wl_ref, b_ref, wr_ref, o_ref):
    summed = s_ref[0] + s_ref[1]
    cnt = c_ref[0, :, 0:1] + c_ref[1, :, 0:1]
    mean = summed / jnp.maximum(cnt, 1.0)
    h = h_ref[...]
    z = (jnp.dot(mean, wl_ref[...], preferred_element_type=jnp.float32)
         + jnp.dot(h, wr_ref[...], preferred_element_type=jnp.float32)
         + b_ref[...])
    o_ref[...] = jnp.where(z > 0, z, jnp.exp(jnp.minimum(z, 0.0)) - 1.0)


def _tc_dense(sums, cnts, h, W_l, b, W_r):
    return pl.pallas_call(
        _tc_dense_kernel,
        grid=(GRID,),
        in_specs=[
            pl.BlockSpec((NC, ROW_BLK, D), lambda i: (0, i, 0)),
            pl.BlockSpec((NC, ROW_BLK, D), lambda i: (0, i, 0)),
            pl.BlockSpec((ROW_BLK, D), lambda i: (i, 0)),
            pl.BlockSpec((D, D), lambda i: (0, 0)),
            pl.BlockSpec((1, D), lambda i: (0, 0)),
            pl.BlockSpec((D, D), lambda i: (0, 0)),
        ],
        out_specs=pl.BlockSpec((ROW_BLK, D), lambda i: (i, 0)),
        out_shape=jax.ShapeDtypeStruct((N_NODES, D), jnp.float32),
    )(sums, cnts, h, W_l, b, W_r)


def _tc_logsoftmax_kernel(h_ref, o_ref):
    z = h_ref[...]
    m = jnp.max(z, axis=1, keepdims=True)
    lse = m + jnp.log(jnp.sum(jnp.exp(z - m), axis=1, keepdims=True))
    o_ref[...] = z - lse


def _tc_logsoftmax(h):
    return pl.pallas_call(
        _tc_logsoftmax_kernel,
        grid=(GRID,),
        in_specs=[pl.BlockSpec((ROW_BLK, D), lambda i: (i, 0))],
        out_specs=pl.BlockSpec((ROW_BLK, D), lambda i: (i, 0)),
        out_shape=jax.ShapeDtypeStruct((N_NODES, D), jnp.float32),
    )(h)


def kernel(x, edge_index, W_l1, b1, W_r1, W_l2, b2, W_r2):
    src4 = edge_index[0].reshape(NW, HALVES, HCHUNKS, K)
    dst4 = edge_index[1].reshape(NW, HALVES, HCHUNKS, K)
    z128 = jnp.zeros((N_NODES, D), jnp.float32)
    ones128 = jnp.ones((K, D), jnp.float32)

    agg_cnt = _sc_kernel(True)
    agg = _sc_kernel(False)

    sums1, cnts = agg_cnt(x, src4, dst4, z128, ones128)
    h1 = _tc_dense(sums1, cnts, x, W_l1, b1.reshape(1, D), W_r1)
    (sums2,) = agg(h1, src4, dst4, z128, ones128)
    h2 = _tc_dense(sums2, cnts, h1, W_l2, b2.reshape(1, D), W_r2)
    return _tc_logsoftmax(h2)
